# Initial kernel scaffold; baseline (speedup 1.0000x reference)
import jax
import jax.numpy as jnp
from jax.experimental import pallas as pl


def kernel(x, edge_index, edge_attr, batch, W1, b1, W2, b2, Wg1, asrc1, adst1, bg1, Wg2, asrc2, adst2, bg2, Wf1, bf1, Wf2, bf2):
    raise NotImplementedError("write your pallas kernel here")


# speedup vs baseline: 23.9352x; 23.9352x over previous
import functools

import jax
import jax.numpy as jnp
from jax import lax
from jax.experimental import pallas as pl
from jax.experimental.pallas import tpu as pltpu
from jax.experimental.pallas import tpu_sc as plsc

F32 = jnp.float32
I32 = jnp.int32

N = 100000
G = 128
H = 64
NPAD = 100352
BLK = 2048
NB = NPAD // BLK
RPT = NPAD // 16
EPAD = 1703936
CHUNK = 1024
EROWS = EPAD // 128

_MESH = dict(core_axis_name="c", subcore_axis_name="s")
_SC_PARAMS = pltpu.CompilerParams(use_tc_tiling_on_sc=False)


def _zero_vmem(ref, nrows):
    z = jnp.zeros((16,), F32)

    def body(i, _):
        ref[i, :] = z
        return 0

    lax.fori_loop(0, nrows, body, 0)


def _zero_vmem_1d(ref, n16):
    z = jnp.zeros((16,), F32)

    def body(i, _):
        ref[pl.ds(i * 16, 16)] = z
        return 0

    lax.fori_loop(0, n16, body, 0)


def _fill_ones_2d(ref, rows):
    o = jnp.ones((16,), F32)

    def body(i, _):
        r = i // 8
        l = lax.rem(i, 8)
        ref[r, pl.ds(l * 16, 16)] = o
        return 0

    lax.fori_loop(0, rows * 8, body, 0)


def _deg_body(dst2, out, onesv, idxv, tmpv, zv, shared, sem):
    core = lax.axis_index("c")
    sub = lax.axis_index("s")
    wid = core * 16 + sub
    row0 = sub * RPT

    _fill_ones_2d(onesv, 8)
    _zero_vmem_1d(zv, RPT // 16)
    for t in range(4):
        pltpu.sync_copy(zv.at[pl.ds(0, RPT // 4)],
                        shared.at[pl.ds(row0 + t * (RPT // 4), RPT // 4)])
    plsc.subcore_barrier()

    def chunk(j, _):
        brow = wid * (EPAD // 32 // 128) + j * 8
        pltpu.sync_copy(dst2.at[pl.ds(brow, 8)], idxv)
        descs = [
            pltpu.async_copy(onesv.at[k], shared.at[idxv.at[k]], sem,
                             add=True)
            for k in range(8)
        ]
        for d in descs:
            d.wait()
        return 0

    lax.fori_loop(0, EPAD // 32 // CHUNK, chunk, 0)
    plsc.subcore_barrier()
    pltpu.sync_copy(shared.at[pl.ds(row0, RPT)], tmpv)
    pltpu.sync_copy(tmpv, out.at[core, pl.ds(row0, RPT)])


@functools.partial(
    pl.kernel,
    out_type=jax.ShapeDtypeStruct((2, NPAD), F32),
    mesh=plsc.VectorSubcoreMesh(**_MESH),
    compiler_params=_SC_PARAMS,
    scratch_types=[
        pltpu.VMEM((8, 128), F32),
        pltpu.VMEM((8, 128), I32),
        pltpu.VMEM((RPT,), F32),
        pltpu.VMEM((RPT,), F32),
        pltpu.VMEM_SHARED((NPAD,), F32),
        pltpu.SemaphoreType.DMA,
    ],
)
def _k_deg(dst2, out, onesv, idxv, tmpv, zv, shared, sem):
    _deg_body(dst2, out, onesv, idxv, tmpv, zv, shared, sem)


def _rows_body(with_ex, ytab, src2, dst2, ex2, out, srcv, dstv, adjv, rowsv,
               exv, zv, tmpv, shared, gsem, ssem):
    core = lax.axis_index("c")
    sub = lax.axis_index("s")
    row0 = sub * RPT
    _zero_vmem(zv, RPT // 32)

    for rnd in range(2):
        col = core + 2 * rnd
        off = col * NPAD

        def zpiece(t, _):
            pltpu.sync_copy(
                zv, shared.at[pl.ds(row0 + t * (RPT // 32), RPT // 32)])
            return 0

        lax.fori_loop(0, 32, zpiece, 0)
        plsc.subcore_barrier()

        def chunk(j, _):
            brow = sub * (EPAD // 16 // 128) + j * 8
            pltpu.sync_copy(src2.at[pl.ds(brow, 8)], srcv)
            pltpu.sync_copy(dst2.at[pl.ds(brow, 8)], dstv)

            def adj(i, _):
                r = i // 8
                l = lax.rem(i, 8)
                adjv[r, pl.ds(l * 16, 16)] = (
                    srcv[r, pl.ds(l * 16, 16)] + off)
                return 0

            lax.fori_loop(0, 64, adj, 0)
            gd = [
                pltpu.async_copy(ytab.at[adjv.at[k]],
                                 rowsv.at[pl.ds(k * 128, 128)], gsem)
                for k in range(8)
            ]
            for d in gd:
                d.wait()
            if with_ex:
                pltpu.sync_copy(ex2.at[pl.ds(brow, 8)], exv)

                def mul(i, _):
                    r = i // 8
                    l = lax.rem(i, 8)
                    e16 = exv[r, pl.ds(l * 16, 16)]
                    base = r * 128 + l * 16
                    for t in range(16):
                        rowsv[base + t, :] = rowsv[base + t, :] * e16[t]
                    return 0

                lax.fori_loop(0, 64, mul, 0)
            sd = [
                pltpu.async_copy(rowsv.at[pl.ds(k * 128, 128)],
                                 shared.at[dstv.at[k]], ssem, add=True)
                for k in range(8)
            ]
            for d in sd:
                d.wait()
            return 0

        lax.fori_loop(0, EPAD // 16 // CHUNK, chunk, 0)
        plsc.subcore_barrier()

        def dpiece(p, _):
            r0 = row0 + p * (RPT // 32)
            pltpu.sync_copy(shared.at[pl.ds(r0, RPT // 32)], tmpv)
            pltpu.sync_copy(tmpv, out.at[pl.ds(off + r0, RPT // 32)])
            return 0

        lax.fori_loop(0, 32, dpiece, 0)
        plsc.subcore_barrier()


def _make_rows(with_ex):
    scratch = [
        pltpu.VMEM((8, 128), I32),
        pltpu.VMEM((8, 128), I32),
        pltpu.VMEM((8, 128), I32),
        pltpu.VMEM((CHUNK, 16), F32),
        pltpu.VMEM((8, 128), F32),
        pltpu.VMEM((RPT // 32, 16), F32),
        pltpu.VMEM((RPT // 32, 16), F32),
        pltpu.VMEM_SHARED((NPAD, 16), F32),
        pltpu.SemaphoreType.DMA,
        pltpu.SemaphoreType.DMA,
    ]
    if with_ex:
        @functools.partial(
            pl.kernel,
            out_type=jax.ShapeDtypeStruct((4 * NPAD, 16), F32),
            mesh=plsc.VectorSubcoreMesh(**_MESH),
    compiler_params=_SC_PARAMS,
            scratch_types=scratch,
        )
        def k(ytab, src2, dst2, ex2, out, *s):
            _rows_body(True, ytab, src2, dst2, ex2, out, *s)
    else:
        @functools.partial(
            pl.kernel,
            out_type=jax.ShapeDtypeStruct((4 * NPAD, 16), F32),
            mesh=plsc.VectorSubcoreMesh(**_MESH),
    compiler_params=_SC_PARAMS,
            scratch_types=scratch,
        )
        def k(ytab, src2, dst2, out, *s):
            _rows_body(False, ytab, src2, dst2, None, out, *s)
    return k


_k_rows = _make_rows(False)
_k_rows_ex = _make_rows(True)


def _gatsc_body(als, ald, cs, src2, dst2, ex2, dpart, srcv, dstv, asv, adv,
                csv, exv, tmpv, zv, shared, gsem, ssem):
    core = lax.axis_index("c")
    sub = lax.axis_index("s")
    wid = core * 16 + sub
    row0 = sub * RPT
    _zero_vmem_1d(zv, RPT // 16)

    for t in range(4):
        pltpu.sync_copy(zv.at[pl.ds(0, RPT // 4)],
                        shared.at[pl.ds(row0 + t * (RPT // 4), RPT // 4)])
    plsc.subcore_barrier()

    def chunk(j, _):
        brow = wid * (EPAD // 32 // 128) + j * 8
        pltpu.sync_copy(src2.at[pl.ds(brow, 8)], srcv)
        pltpu.sync_copy(dst2.at[pl.ds(brow, 8)], dstv)
        gd = []
        for k in range(8):
            gd.append(pltpu.async_copy(als.at[srcv.at[k]], asv.at[k],
                                       gsem))
            gd.append(pltpu.async_copy(ald.at[dstv.at[k]], adv.at[k],
                                       gsem))
            gd.append(pltpu.async_copy(cs.at[dstv.at[k]], csv.at[k],
                                       gsem))
        for d in gd:
            d.wait()

        def comp(i, _):
            r = i // 8
            l = lax.rem(i, 8)
            sl = pl.ds(l * 16, 16)
            s = asv[r, sl] + adv[r, sl]
            e = jnp.maximum(s, 0.0) + 0.2 * jnp.minimum(s, 0.0)
            exv[r, sl] = jnp.exp(e - csv[r, sl])
            return 0

        lax.fori_loop(0, 64, comp, 0)
        pltpu.sync_copy(exv, ex2.at[pl.ds(brow, 8)])
        sd = [
            pltpu.async_copy(exv.at[k], shared.at[dstv.at[k]], ssem,
                             add=True)
            for k in range(8)
        ]
        for d in sd:
            d.wait()
        return 0

    lax.fori_loop(0, EPAD // 32 // CHUNK, chunk, 0)
    plsc.subcore_barrier()
    pltpu.sync_copy(shared.at[pl.ds(row0, RPT)], tmpv)
    pltpu.sync_copy(tmpv, dpart.at[core, pl.ds(row0, RPT)])


@functools.partial(
    pl.kernel,
    out_type=(jax.ShapeDtypeStruct((EROWS, 128), F32),
              jax.ShapeDtypeStruct((2, NPAD), F32)),
    mesh=plsc.VectorSubcoreMesh(**_MESH),
    compiler_params=_SC_PARAMS,
    scratch_types=[
        pltpu.VMEM((8, 128), I32),
        pltpu.VMEM((8, 128), I32),
        pltpu.VMEM((8, 128), F32),
        pltpu.VMEM((8, 128), F32),
        pltpu.VMEM((8, 128), F32),
        pltpu.VMEM((8, 128), F32),
        pltpu.VMEM((RPT,), F32),
        pltpu.VMEM((RPT,), F32),
        pltpu.VMEM_SHARED((NPAD,), F32),
        pltpu.SemaphoreType.DMA,
        pltpu.SemaphoreType.DMA,
    ],
)
def _k_gatsc(als, ald, cs, src2, dst2, ex2, dpart, *s):
    _gatsc_body(als, ald, cs, src2, dst2, ex2, dpart, *s)


def _cols_to_h(raw_ref, scale, b_ref):
    hs = []
    for c in range(4):
        hc = jnp.maximum(scale * raw_ref[c] + b_ref[pl.ds(c * 16, 16)], 0.0)
        hs.append(hc)
    return jnp.concatenate(hs, axis=1)


def _prep1_body(x_ref, degp_ref, w_ref, ytab_ref, dinv_ref):
    deg = degp_ref[0] + degp_ref[1]
    dinv = jnp.where(deg > 0.0, lax.rsqrt(jnp.maximum(deg, 1e-30)), 0.0)
    xw = jnp.dot(x_ref[...], w_ref[...], preferred_element_type=F32)
    y = dinv * xw
    for c in range(4):
        ytab_ref[c] = y[:, c * 16:(c + 1) * 16]
    dinv_ref[...] = dinv


_k_prep1 = pl.pallas_call(
    _prep1_body,
    grid=(NB,),
    in_specs=[
        pl.BlockSpec((BLK, 32), lambda i: (i, 0)),
        pl.BlockSpec((2, BLK, 1), lambda i: (0, i, 0)),
        pl.BlockSpec((32, 64), lambda i: (0, 0)),
    ],
    out_specs=[
        pl.BlockSpec((4, BLK, 16), lambda i: (0, i, 0)),
        pl.BlockSpec((BLK, 1), lambda i: (i, 0)),
    ],
    out_shape=[
        jax.ShapeDtypeStruct((4, NPAD, 16), F32),
        jax.ShapeDtypeStruct((NPAD, 1), F32),
    ],
)


def _midgcn_body(raw_ref, dinv_ref, b_ref, w_ref, ytab_ref):
    dinv = dinv_ref[...]
    acc = jnp.zeros((BLK, 64), F32)
    for c in range(4):
        hc = jnp.maximum(dinv * raw_ref[c] + b_ref[pl.ds(c * 16, 16)], 0.0)
        acc = acc + jnp.dot(hc, w_ref[pl.ds(c * 16, 16), :],
                            preferred_element_type=F32)
    y = dinv * acc
    for c in range(4):
        ytab_ref[c] = y[:, c * 16:(c + 1) * 16]


_k_midgcn = pl.pallas_call(
    _midgcn_body,
    grid=(NB,),
    in_specs=[
        pl.BlockSpec((4, BLK, 16), lambda i: (0, i, 0)),
        pl.BlockSpec((BLK, 1), lambda i: (i, 0)),
        pl.BlockSpec((64,), lambda i: (0,)),
        pl.BlockSpec((64, 64), lambda i: (0, 0)),
    ],
    out_specs=pl.BlockSpec((4, BLK, 16), lambda i: (0, i, 0)),
    out_shape=jax.ShapeDtypeStruct((4, NPAD, 16), F32),
)


def _gatprep_body(scale_kind, raw_ref, s_ref, b_ref, w_ref, asrc_ref,
                  adst_ref, xwtab_ref, als_ref, ald_ref, cs_ref):
    if scale_kind == "dinv":
        scale = s_ref[...]
    else:
        dsum = s_ref[0] + s_ref[1]
        scale = 1.0 / jnp.where(dsum > 0.0, dsum, 1.0)
    acc = jnp.zeros((BLK, 64), F32)
    for c in range(4):
        hc = jnp.maximum(scale * raw_ref[c] + b_ref[pl.ds(c * 16, 16)], 0.0)
        acc = acc + jnp.dot(hc, w_ref[pl.ds(c * 16, 16), :],
                            preferred_element_type=F32)
    for c in range(4):
        xwtab_ref[c] = acc[:, c * 16:(c + 1) * 16]
    als = jnp.dot(acc, asrc_ref[...], preferred_element_type=F32)
    ald = jnp.dot(acc, adst_ref[...], preferred_element_type=F32)
    s = als + ald
    cs = jnp.maximum(s, 0.0) + 0.2 * jnp.minimum(s, 0.0)
    als_ref[...] = als
    ald_ref[...] = ald
    cs_ref[...] = cs


def _make_gatprep(scale_kind):
    sspec = (pl.BlockSpec((BLK, 1), lambda i: (i, 0)) if scale_kind == "dinv"
             else pl.BlockSpec((2, BLK, 1), lambda i: (0, i, 0)))
    return pl.pallas_call(
        functools.partial(_gatprep_body, scale_kind),
        grid=(NB,),
        in_specs=[
            pl.BlockSpec((4, BLK, 16), lambda i: (0, i, 0)),
            sspec,
            pl.BlockSpec((64,), lambda i: (0,)),
            pl.BlockSpec((64, 64), lambda i: (0, 0)),
            pl.BlockSpec((64, 1), lambda i: (0, 0)),
            pl.BlockSpec((64, 1), lambda i: (0, 0)),
        ],
        out_specs=[
            pl.BlockSpec((4, BLK, 16), lambda i: (0, i, 0)),
            pl.BlockSpec((BLK, 1), lambda i: (i, 0)),
            pl.BlockSpec((BLK, 1), lambda i: (i, 0)),
            pl.BlockSpec((BLK, 1), lambda i: (i, 0)),
        ],
        out_shape=[
            jax.ShapeDtypeStruct((4, NPAD, 16), F32),
            jax.ShapeDtypeStruct((NPAD, 1), F32),
            jax.ShapeDtypeStruct((NPAD, 1), F32),
            jax.ShapeDtypeStruct((NPAD, 1), F32),
        ],
    )


_k_gatprep_dinv = _make_gatprep("dinv")
_k_gatprep_denom = _make_gatprep("denom")


def _final_body(raw_ref, dp_ref, b_ref, batch_ref, wf1_ref, bf1_ref, wf2_ref,
                bf2_ref, out_ref, sums_ref, cnt_ref):
    i = pl.program_id(0)

    @pl.when(i == 0)
    def _():
        sums_ref[...] = jnp.zeros((G, 64), F32)
        cnt_ref[...] = jnp.zeros((G, 1), F32)

    dsum = dp_ref[0] + dp_ref[1]
    scale = 1.0 / jnp.where(dsum > 0.0, dsum, 1.0)
    hs = []
    for c in range(4):
        hs.append(jnp.maximum(scale * raw_ref[c] + b_ref[pl.ds(c * 16, 16)],
                              0.0))
    h = jnp.concatenate(hs, axis=1)
    gids = lax.broadcasted_iota(I32, (1, G), 1)
    oh = (batch_ref[...] == gids).astype(F32)
    sums_ref[...] += lax.dot_general(oh, h, (((0,), (0,)), ((), ())),
                                     preferred_element_type=F32)
    cnt_ref[...] += lax.dot_general(oh, jnp.ones((BLK, 1), F32),
                                    (((0,), (0,)), ((), ())),
                                    preferred_element_type=F32)

    @pl.when(i == NB - 1)
    def _():
        pooled = sums_ref[...] / jnp.maximum(cnt_ref[...], 1.0)
        hf = jnp.maximum(
            jnp.dot(pooled, wf1_ref[...], preferred_element_type=F32)
            + bf1_ref[...], 0.0)
        out_ref[...] = (jnp.dot(hf, wf2_ref[...], preferred_element_type=F32)
                        + bf2_ref[...])


_k_final = pl.pallas_call(
    _final_body,
    grid=(NB,),
    in_specs=[
        pl.BlockSpec((4, BLK, 16), lambda i: (0, i, 0)),
        pl.BlockSpec((2, BLK, 1), lambda i: (0, i, 0)),
        pl.BlockSpec((64,), lambda i: (0,)),
        pl.BlockSpec((BLK, 1), lambda i: (i, 0)),
        pl.BlockSpec((64, 32), lambda i: (0, 0)),
        pl.BlockSpec((32,), lambda i: (0,)),
        pl.BlockSpec((32, 1), lambda i: (0, 0)),
        pl.BlockSpec((1,), lambda i: (0,)),
    ],
    out_specs=pl.BlockSpec((G, 1), lambda i: (0, 0)),
    out_shape=jax.ShapeDtypeStruct((G, 1), F32),
    scratch_shapes=[
        pltpu.VMEM((G, 64), F32),
        pltpu.VMEM((G, 1), F32),
    ],
)


def kernel(x, edge_index, edge_attr, batch, W1, b1, W2, b2, Wg1, asrc1,
           adst1, bg1, Wg2, asrc2, adst2, bg2, Wf1, bf1, Wf2, bf2):
    del edge_attr
    loop = jnp.arange(N, dtype=I32)
    npad_e = EPAD - edge_index.shape[1] - N
    padi = jnp.full((npad_e,), N, I32)
    src2 = jnp.concatenate([edge_index[0], loop, padi]).reshape(EROWS, 128)
    dst2 = jnp.concatenate([edge_index[1], loop, padi]).reshape(EROWS, 128)
    x_pad = jnp.zeros((NPAD, 32), F32).at[:N].set(x)
    batch_pad = jnp.full((NPAD, 1), -1, I32).at[:N, 0].set(batch)

    degp = _k_deg(dst2)
    ytab1, dinv = _k_prep1(x_pad, degp.reshape(2, NPAD, 1), W1)
    raw1 = _k_rows(ytab1.reshape(4 * NPAD, 16), src2, dst2)
    ytab2 = _k_midgcn(raw1.reshape(4, NPAD, 16), dinv, b1, W2)
    raw2 = _k_rows(ytab2.reshape(4 * NPAD, 16), src2, dst2)
    xwtab3, als3, ald3, cs3 = _k_gatprep_dinv(
        raw2.reshape(4, NPAD, 16), dinv, b2, Wg1, asrc1.reshape(64, 1),
        adst1.reshape(64, 1))
    ex3, dp3 = _k_gatsc(als3.reshape(NPAD), ald3.reshape(NPAD),
                        cs3.reshape(NPAD), src2, dst2)
    raw3 = _k_rows_ex(xwtab3.reshape(4 * NPAD, 16), src2, dst2, ex3)
    xwtab4, als4, ald4, cs4 = _k_gatprep_denom(
        raw3.reshape(4, NPAD, 16), dp3.reshape(2, NPAD, 1), bg1, Wg2,
        asrc2.reshape(64, 1), adst2.reshape(64, 1))
    ex4, dp4 = _k_gatsc(als4.reshape(NPAD), ald4.reshape(NPAD),
                        cs4.reshape(NPAD), src2, dst2)
    raw4 = _k_rows_ex(xwtab4.reshape(4 * NPAD, 16), src2, dst2, ex4)
    out = _k_final(raw4.reshape(4, NPAD, 16), dp4.reshape(2, NPAD, 1), bg2,
                   batch_pad, Wf1, bf1, Wf2, bf2)
    return out.reshape(G)


# speedup vs baseline: 23.9823x; 1.0020x over previous
import functools

import jax
import jax.numpy as jnp
from jax import lax
from jax.experimental import pallas as pl
from jax.experimental.pallas import tpu as pltpu
from jax.experimental.pallas import tpu_sc as plsc

F32 = jnp.float32
I32 = jnp.int32

N = 100000
G = 128
H = 64
NPAD = 100352
BLK = 2048
NB = NPAD // BLK
RPT = NPAD // 16
EPAD = 1703936
CHUNK = 1024
EPC = EPAD // 16
EPW = EPAD // 32

_MESH = dict(core_axis_name="c", subcore_axis_name="s")
_SC_PARAMS = pltpu.CompilerParams(use_tc_tiling_on_sc=False)


def _zero_vmem(ref, nrows):
    z = jnp.zeros((16,), F32)

    def body(i, _):
        ref[i, :] = z
        return 0

    lax.fori_loop(0, nrows, body, 0)


def _zero_vmem_1d(ref, n16):
    z = jnp.zeros((16,), F32)

    def body(i, _):
        ref[pl.ds(i * 16, 16)] = z
        return 0

    lax.fori_loop(0, n16, body, 0)


def _fill_ones_1d(ref, n16):
    o = jnp.ones((16,), F32)

    def body(i, _):
        ref[pl.ds(i * 16, 16)] = o
        return 0

    lax.fori_loop(0, n16, body, 0)


def _deg_body(dst1, out, onesv, idxv, zv, shared, sem):
    core = lax.axis_index("c")
    sub = lax.axis_index("s")
    wid = core * 16 + sub
    row0 = sub * RPT

    _fill_ones_1d(onesv, CHUNK // 16)
    _zero_vmem_1d(zv, RPT // 16)
    pltpu.sync_copy(zv, shared.at[pl.ds(row0, RPT)])
    plsc.subcore_barrier()

    def chunk(j, _):
        e0 = wid * EPW + j * CHUNK
        pltpu.sync_copy(dst1.at[pl.ds(e0, CHUNK)], idxv)
        pltpu.async_copy(onesv, shared.at[idxv], sem, add=True).wait()
        return 0

    lax.fori_loop(0, EPW // CHUNK, chunk, 0)
    plsc.subcore_barrier()
    pltpu.async_copy(shared.at[pl.ds(row0, RPT)],
                     out.at[core, pl.ds(row0, RPT)], sem).wait()


@functools.partial(
    pl.kernel,
    out_type=jax.ShapeDtypeStruct((2, NPAD), F32),
    mesh=plsc.VectorSubcoreMesh(**_MESH),
    compiler_params=_SC_PARAMS,
    scratch_types=[
        pltpu.VMEM((CHUNK,), F32),
        pltpu.VMEM((CHUNK,), I32),
        pltpu.VMEM((RPT,), F32),
        pltpu.VMEM_SHARED((NPAD,), F32),
        pltpu.SemaphoreType.DMA,
    ],
)
def _k_deg(dst1, out, onesv, idxv, zv, shared, sem):
    _deg_body(dst1, out, onesv, idxv, zv, shared, sem)


def _rows_body(with_ex, ytab, src1, dst1, ex1, out, srcv, dstv, adjv, rowsv,
               exv, zv, shared, gsem, ssem):
    core = lax.axis_index("c")
    sub = lax.axis_index("s")
    row0 = sub * RPT

    _zero_vmem(zv, RPT // 32)

    for rnd in range(2):
        col = core + 2 * rnd
        off = col * NPAD
        zd = [
            pltpu.async_copy(
                zv, shared.at[pl.ds(row0 + t * (RPT // 32), RPT // 32)],
                gsem)
            for t in range(32)
        ]
        for d in zd:
            d.wait()
        plsc.subcore_barrier()

        def chunk(j, _):
            e0 = sub * EPC + j * CHUNK
            pltpu.sync_copy(src1.at[pl.ds(e0, CHUNK)], srcv)
            pltpu.sync_copy(dst1.at[pl.ds(e0, CHUNK)], dstv)

            def adj(i, _):
                sl = pl.ds(i * 16, 16)
                adjv[sl] = srcv[sl] + off
                return 0

            lax.fori_loop(0, CHUNK // 16, adj, 0)
            pltpu.async_copy(ytab.at[adjv], rowsv, gsem).wait()
            if with_ex:
                pltpu.sync_copy(ex1.at[pl.ds(e0, CHUNK)], exv)

                def mul(i, _):
                    e16 = exv[pl.ds(i * 16, 16)]
                    base = i * 16
                    for t in range(16):
                        rowsv[base + t, :] = rowsv[base + t, :] * e16[t]
                    return 0

                lax.fori_loop(0, CHUNK // 16, mul, 0)
            pltpu.async_copy(rowsv, shared.at[dstv], ssem, add=True).wait()
            return 0

        lax.fori_loop(0, EPC // CHUNK, chunk, 0)
        plsc.subcore_barrier()
        pltpu.async_copy(shared.at[pl.ds(row0, RPT)],
                         out.at[pl.ds(off + row0, RPT)], gsem).wait()
        plsc.subcore_barrier()


def _make_rows(with_ex):
    scratch = [
        pltpu.VMEM((CHUNK,), I32),
        pltpu.VMEM((CHUNK,), I32),
        pltpu.VMEM((CHUNK,), I32),
        pltpu.VMEM((CHUNK, 16), F32),
        pltpu.VMEM((CHUNK,), F32),
        pltpu.VMEM((RPT // 32, 16), F32),
        pltpu.VMEM_SHARED((NPAD, 16), F32),
        pltpu.SemaphoreType.DMA,
        pltpu.SemaphoreType.DMA,
    ]
    if with_ex:
        @functools.partial(
            pl.kernel,
            out_type=jax.ShapeDtypeStruct((4 * NPAD, 16), F32),
            mesh=plsc.VectorSubcoreMesh(**_MESH),
            compiler_params=_SC_PARAMS,
            scratch_types=scratch,
        )
        def k(ytab, src1, dst1, ex1, out, *s):
            _rows_body(True, ytab, src1, dst1, ex1, out, *s)
    else:
        @functools.partial(
            pl.kernel,
            out_type=jax.ShapeDtypeStruct((4 * NPAD, 16), F32),
            mesh=plsc.VectorSubcoreMesh(**_MESH),
            compiler_params=_SC_PARAMS,
            scratch_types=scratch,
        )
        def k(ytab, src1, dst1, out, *s):
            _rows_body(False, ytab, src1, dst1, None, out, *s)
    return k


_k_rows = _make_rows(False)
_k_rows_ex = _make_rows(True)


def _gatsc_body(als, ald, cs, src1, dst1, ex1, dpart, srcv, dstv, asv, adv,
                csv, exv, zv, shared, gsem, ssem):
    core = lax.axis_index("c")
    sub = lax.axis_index("s")
    wid = core * 16 + sub
    row0 = sub * RPT
    _zero_vmem_1d(zv, RPT // 16)
    pltpu.sync_copy(zv, shared.at[pl.ds(row0, RPT)])
    plsc.subcore_barrier()

    def chunk(j, _):
        e0 = wid * EPW + j * CHUNK
        pltpu.sync_copy(src1.at[pl.ds(e0, CHUNK)], srcv)
        pltpu.sync_copy(dst1.at[pl.ds(e0, CHUNK)], dstv)
        gd = [
            pltpu.async_copy(als.at[srcv], asv, gsem),
            pltpu.async_copy(ald.at[dstv], adv, gsem),
            pltpu.async_copy(cs.at[dstv], csv, gsem),
        ]
        for d in gd:
            d.wait()

        def comp(i, _):
            sl = pl.ds(i * 16, 16)
            s = asv[sl] + adv[sl]
            e = jnp.maximum(s, 0.0) + 0.2 * jnp.minimum(s, 0.0)
            exv[sl] = jnp.exp(e - csv[sl])
            return 0

        lax.fori_loop(0, CHUNK // 16, comp, 0)
        pltpu.sync_copy(exv, ex1.at[pl.ds(e0, CHUNK)])
        pltpu.async_copy(exv, shared.at[dstv], ssem, add=True).wait()
        return 0

    lax.fori_loop(0, EPW // CHUNK, chunk, 0)
    plsc.subcore_barrier()
    pltpu.async_copy(shared.at[pl.ds(row0, RPT)],
                     dpart.at[core, pl.ds(row0, RPT)], gsem).wait()


@functools.partial(
    pl.kernel,
    out_type=(jax.ShapeDtypeStruct((EPAD,), F32),
              jax.ShapeDtypeStruct((2, NPAD), F32)),
    mesh=plsc.VectorSubcoreMesh(**_MESH),
    compiler_params=_SC_PARAMS,
    scratch_types=[
        pltpu.VMEM((CHUNK,), I32),
        pltpu.VMEM((CHUNK,), I32),
        pltpu.VMEM((CHUNK,), F32),
        pltpu.VMEM((CHUNK,), F32),
        pltpu.VMEM((CHUNK,), F32),
        pltpu.VMEM((CHUNK,), F32),
        pltpu.VMEM((RPT,), F32),
        pltpu.VMEM_SHARED((NPAD,), F32),
        pltpu.SemaphoreType.DMA,
        pltpu.SemaphoreType.DMA,
    ],
)
def _k_gatsc(als, ald, cs, src1, dst1, ex1, dpart, *s):
    _gatsc_body(als, ald, cs, src1, dst1, ex1, dpart, *s)


def _prep1_body(x_ref, degp_ref, w_ref, ytab_ref, dinv_ref):
    deg = degp_ref[0] + degp_ref[1]
    dinv = jnp.where(deg > 0.0, lax.rsqrt(jnp.maximum(deg, 1e-30)), 0.0)
    xw = jnp.dot(x_ref[...], w_ref[...], preferred_element_type=F32)
    y = dinv * xw
    for c in range(4):
        ytab_ref[c] = y[:, c * 16:(c + 1) * 16]
    dinv_ref[...] = dinv


_k_prep1 = pl.pallas_call(
    _prep1_body,
    grid=(NB,),
    in_specs=[
        pl.BlockSpec((BLK, 32), lambda i: (i, 0)),
        pl.BlockSpec((2, BLK, 1), lambda i: (0, i, 0)),
        pl.BlockSpec((32, 64), lambda i: (0, 0)),
    ],
    out_specs=[
        pl.BlockSpec((4, BLK, 16), lambda i: (0, i, 0)),
        pl.BlockSpec((BLK, 1), lambda i: (i, 0)),
    ],
    out_shape=[
        jax.ShapeDtypeStruct((4, NPAD, 16), F32),
        jax.ShapeDtypeStruct((NPAD, 1), F32),
    ],
)


def _midgcn_body(raw_ref, dinv_ref, b_ref, w_ref, ytab_ref):
    dinv = dinv_ref[...]
    acc = jnp.zeros((BLK, 64), F32)
    for c in range(4):
        hc = jnp.maximum(dinv * raw_ref[c] + b_ref[pl.ds(c * 16, 16)], 0.0)
        acc = acc + jnp.dot(hc, w_ref[pl.ds(c * 16, 16), :],
                            preferred_element_type=F32)
    y = dinv * acc
    for c in range(4):
        ytab_ref[c] = y[:, c * 16:(c + 1) * 16]


_k_midgcn = pl.pallas_call(
    _midgcn_body,
    grid=(NB,),
    in_specs=[
        pl.BlockSpec((4, BLK, 16), lambda i: (0, i, 0)),
        pl.BlockSpec((BLK, 1), lambda i: (i, 0)),
        pl.BlockSpec((64,), lambda i: (0,)),
        pl.BlockSpec((64, 64), lambda i: (0, 0)),
    ],
    out_specs=pl.BlockSpec((4, BLK, 16), lambda i: (0, i, 0)),
    out_shape=jax.ShapeDtypeStruct((4, NPAD, 16), F32),
)


def _gatprep_body(scale_kind, raw_ref, s_ref, b_ref, w_ref, asrc_ref,
                  adst_ref, xwtab_ref, als_ref, ald_ref, cs_ref):
    if scale_kind == "dinv":
        scale = s_ref[...]
    else:
        dsum = s_ref[0] + s_ref[1]
        scale = 1.0 / jnp.where(dsum > 0.0, dsum, 1.0)
    acc = jnp.zeros((BLK, 64), F32)
    for c in range(4):
        hc = jnp.maximum(scale * raw_ref[c] + b_ref[pl.ds(c * 16, 16)], 0.0)
        acc = acc + jnp.dot(hc, w_ref[pl.ds(c * 16, 16), :],
                            preferred_element_type=F32)
    for c in range(4):
        xwtab_ref[c] = acc[:, c * 16:(c + 1) * 16]
    als = jnp.dot(acc, asrc_ref[...], preferred_element_type=F32)
    ald = jnp.dot(acc, adst_ref[...], preferred_element_type=F32)
    s = als + ald
    cs = jnp.maximum(s, 0.0) + 0.2 * jnp.minimum(s, 0.0)
    als_ref[...] = als
    ald_ref[...] = ald
    cs_ref[...] = cs


def _make_gatprep(scale_kind):
    sspec = (pl.BlockSpec((BLK, 1), lambda i: (i, 0)) if scale_kind == "dinv"
             else pl.BlockSpec((2, BLK, 1), lambda i: (0, i, 0)))
    return pl.pallas_call(
        functools.partial(_gatprep_body, scale_kind),
        grid=(NB,),
        in_specs=[
            pl.BlockSpec((4, BLK, 16), lambda i: (0, i, 0)),
            sspec,
            pl.BlockSpec((64,), lambda i: (0,)),
            pl.BlockSpec((64, 64), lambda i: (0, 0)),
            pl.BlockSpec((64, 1), lambda i: (0, 0)),
            pl.BlockSpec((64, 1), lambda i: (0, 0)),
        ],
        out_specs=[
            pl.BlockSpec((4, BLK, 16), lambda i: (0, i, 0)),
            pl.BlockSpec((BLK, 1), lambda i: (i, 0)),
            pl.BlockSpec((BLK, 1), lambda i: (i, 0)),
            pl.BlockSpec((BLK, 1), lambda i: (i, 0)),
        ],
        out_shape=[
            jax.ShapeDtypeStruct((4, NPAD, 16), F32),
            jax.ShapeDtypeStruct((NPAD, 1), F32),
            jax.ShapeDtypeStruct((NPAD, 1), F32),
            jax.ShapeDtypeStruct((NPAD, 1), F32),
        ],
    )


_k_gatprep_dinv = _make_gatprep("dinv")
_k_gatprep_denom = _make_gatprep("denom")


def _final_body(raw_ref, dp_ref, b_ref, batch_ref, wf1_ref, bf1_ref, wf2_ref,
                bf2_ref, out_ref, sums_ref, cnt_ref):
    i = pl.program_id(0)

    @pl.when(i == 0)
    def _():
        sums_ref[...] = jnp.zeros((G, 64), F32)
        cnt_ref[...] = jnp.zeros((G, 1), F32)

    dsum = dp_ref[0] + dp_ref[1]
    scale = 1.0 / jnp.where(dsum > 0.0, dsum, 1.0)
    hs = []
    for c in range(4):
        hs.append(jnp.maximum(scale * raw_ref[c] + b_ref[pl.ds(c * 16, 16)],
                              0.0))
    h = jnp.concatenate(hs, axis=1)
    gids = lax.broadcasted_iota(I32, (1, G), 1)
    oh = (batch_ref[...] == gids).astype(F32)
    sums_ref[...] += lax.dot_general(oh, h, (((0,), (0,)), ((), ())),
                                     preferred_element_type=F32)
    cnt_ref[...] += lax.dot_general(oh, jnp.ones((BLK, 1), F32),
                                    (((0,), (0,)), ((), ())),
                                    preferred_element_type=F32)

    @pl.when(i == NB - 1)
    def _():
        pooled = sums_ref[...] / jnp.maximum(cnt_ref[...], 1.0)
        hf = jnp.maximum(
            jnp.dot(pooled, wf1_ref[...], preferred_element_type=F32)
            + bf1_ref[...], 0.0)
        out_ref[...] = (jnp.dot(hf, wf2_ref[...], preferred_element_type=F32)
                        + bf2_ref[...])


_k_final = pl.pallas_call(
    _final_body,
    grid=(NB,),
    in_specs=[
        pl.BlockSpec((4, BLK, 16), lambda i: (0, i, 0)),
        pl.BlockSpec((2, BLK, 1), lambda i: (0, i, 0)),
        pl.BlockSpec((64,), lambda i: (0,)),
        pl.BlockSpec((BLK, 1), lambda i: (i, 0)),
        pl.BlockSpec((64, 32), lambda i: (0, 0)),
        pl.BlockSpec((32,), lambda i: (0,)),
        pl.BlockSpec((32, 1), lambda i: (0, 0)),
        pl.BlockSpec((1,), lambda i: (0,)),
    ],
    out_specs=pl.BlockSpec((G, 1), lambda i: (0, 0)),
    out_shape=jax.ShapeDtypeStruct((G, 1), F32),
    scratch_shapes=[
        pltpu.VMEM((G, 64), F32),
        pltpu.VMEM((G, 1), F32),
    ],
)


def kernel(x, edge_index, edge_attr, batch, W1, b1, W2, b2, Wg1, asrc1,
           adst1, bg1, Wg2, asrc2, adst2, bg2, Wf1, bf1, Wf2, bf2):
    del edge_attr
    loop = jnp.arange(N, dtype=I32)
    npad_e = EPAD - edge_index.shape[1] - N
    padi = jnp.full((npad_e,), N, I32)
    src1 = jnp.concatenate([edge_index[0], loop, padi])
    dst1 = jnp.concatenate([edge_index[1], loop, padi])
    x_pad = jnp.zeros((NPAD, 32), F32).at[:N].set(x)
    batch_pad = jnp.full((NPAD, 1), -1, I32).at[:N, 0].set(batch)

    degp = _k_deg(dst1)
    ytab1, dinv = _k_prep1(x_pad, degp.reshape(2, NPAD, 1), W1)
    raw1 = _k_rows(ytab1.reshape(4 * NPAD, 16), src1, dst1)
    ytab2 = _k_midgcn(raw1.reshape(4, NPAD, 16), dinv, b1, W2)
    raw2 = _k_rows(ytab2.reshape(4 * NPAD, 16), src1, dst1)
    xwtab3, als3, ald3, cs3 = _k_gatprep_dinv(
        raw2.reshape(4, NPAD, 16), dinv, b2, Wg1, asrc1.reshape(64, 1),
        adst1.reshape(64, 1))
    ex3, dp3 = _k_gatsc(als3.reshape(NPAD), ald3.reshape(NPAD),
                        cs3.reshape(NPAD), src1, dst1)
    raw3 = _k_rows_ex(xwtab3.reshape(4 * NPAD, 16), src1, dst1, ex3)
    xwtab4, als4, ald4, cs4 = _k_gatprep_denom(
        raw3.reshape(4, NPAD, 16), dp3.reshape(2, NPAD, 1), bg1, Wg2,
        asrc2.reshape(64, 1), adst2.reshape(64, 1))
    ex4, dp4 = _k_gatsc(als4.reshape(NPAD), ald4.reshape(NPAD),
                        cs4.reshape(NPAD), src1, dst1)
    raw4 = _k_rows_ex(xwtab4.reshape(4 * NPAD, 16), src1, dst1, ex4)
    out = _k_final(raw4.reshape(4, NPAD, 16), dp4.reshape(2, NPAD, 1), bg2,
                   batch_pad, Wf1, bf1, Wf2, bf2)
    return out.reshape(G)


# speedup vs baseline: 26.8008x; 1.1175x over previous
import functools

import jax
import jax.numpy as jnp
from jax import lax
from jax.experimental import pallas as pl
from jax.experimental.pallas import tpu as pltpu
from jax.experimental.pallas import tpu_sc as plsc

F32 = jnp.float32
I32 = jnp.int32

N = 100000
G = 128
H = 64
NPAD = 100352
BLK = 2048
NB = NPAD // BLK
RPT = NPAD // 16
EPAD = 1605632
CHUNK = 1024
EPC = EPAD // 16
EPW = EPAD // 32

_MESH = dict(core_axis_name="c", subcore_axis_name="s")
_SC_PARAMS = pltpu.CompilerParams(use_tc_tiling_on_sc=False)


def _zero_vmem(ref, nrows):
    z = jnp.zeros((16,), F32)

    def body(i, _):
        ref[i, :] = z
        return 0

    lax.fori_loop(0, nrows, body, 0)


def _zero_vmem_1d(ref, n16):
    z = jnp.zeros((16,), F32)

    def body(i, _):
        ref[pl.ds(i * 16, 16)] = z
        return 0

    lax.fori_loop(0, n16, body, 0)


def _fill_ones_1d(ref, n16):
    o = jnp.ones((16,), F32)

    def body(i, _):
        ref[pl.ds(i * 16, 16)] = o
        return 0

    lax.fori_loop(0, n16, body, 0)


def _deg_body(dst1, out, onesv, idxv, zv, shared, sem):
    core = lax.axis_index("c")
    sub = lax.axis_index("s")
    wid = core * 16 + sub
    row0 = sub * RPT

    _fill_ones_1d(onesv, CHUNK // 16)
    _zero_vmem_1d(zv, RPT // 16)
    pltpu.sync_copy(zv, shared.at[pl.ds(row0, RPT)])
    plsc.subcore_barrier()

    def chunk(j, _):
        e0 = wid * EPW + j * CHUNK
        pltpu.sync_copy(dst1.at[pl.ds(e0, CHUNK)], idxv)
        pltpu.async_copy(onesv, shared.at[idxv], sem, add=True).wait()
        return 0

    lax.fori_loop(0, EPW // CHUNK, chunk, 0)
    plsc.subcore_barrier()
    pltpu.async_copy(shared.at[pl.ds(row0, RPT)],
                     out.at[core, pl.ds(row0, RPT)], sem).wait()


@functools.partial(
    pl.kernel,
    out_type=jax.ShapeDtypeStruct((2, NPAD), F32),
    mesh=plsc.VectorSubcoreMesh(**_MESH),
    compiler_params=_SC_PARAMS,
    scratch_types=[
        pltpu.VMEM((CHUNK,), F32),
        pltpu.VMEM((CHUNK,), I32),
        pltpu.VMEM((RPT,), F32),
        pltpu.VMEM_SHARED((NPAD,), F32),
        pltpu.SemaphoreType.DMA,
    ],
)
def _k_deg(dst1, out, onesv, idxv, zv, shared, sem):
    _deg_body(dst1, out, onesv, idxv, zv, shared, sem)


def _rows_body(with_ex, rounds, ytab, src1, dst1, ex1, out, srcv, dstv, adjv,
               rowsv, exv, zv, shared, gsem, ssem):
    core = lax.axis_index("c")
    sub = lax.axis_index("s")
    row0 = sub * RPT

    _zero_vmem(zv, RPT // 32)

    for rnd in range(rounds):
        col = core + 2 * rnd
        off = col * NPAD
        zd = [
            pltpu.async_copy(
                zv, shared.at[pl.ds(row0 + t * (RPT // 32), RPT // 32)],
                gsem)
            for t in range(32)
        ]
        for d in zd:
            d.wait()
        plsc.subcore_barrier()

        def chunk(j, _):
            e0 = sub * EPC + j * CHUNK
            pltpu.sync_copy(src1.at[pl.ds(e0, CHUNK)], srcv)
            pltpu.sync_copy(dst1.at[pl.ds(e0, CHUNK)], dstv)

            def adj(i, _):
                sl = pl.ds(i * 16, 16)
                adjv[sl] = srcv[sl] + off
                return 0

            lax.fori_loop(0, CHUNK // 16, adj, 0)
            pltpu.async_copy(ytab.at[adjv], rowsv, gsem).wait()
            if with_ex:
                pltpu.sync_copy(ex1.at[pl.ds(e0, CHUNK)], exv)

                def mul(i, _):
                    e16 = exv[pl.ds(i * 16, 16)]
                    base = i * 16
                    for t in range(16):
                        rowsv[base + t, :] = rowsv[base + t, :] * e16[t]
                    return 0

                lax.fori_loop(0, CHUNK // 16, mul, 0)
            pltpu.async_copy(rowsv, shared.at[dstv], ssem, add=True).wait()
            return 0

        lax.fori_loop(0, EPC // CHUNK, chunk, 0)
        plsc.subcore_barrier()
        pltpu.async_copy(shared.at[pl.ds(row0, RPT)],
                         out.at[pl.ds(off + row0, RPT)], gsem).wait()
        plsc.subcore_barrier()


def _make_rows(with_ex, rounds):
    ncols = 2 * rounds
    scratch = [
        pltpu.VMEM((CHUNK,), I32),
        pltpu.VMEM((CHUNK,), I32),
        pltpu.VMEM((CHUNK,), I32),
        pltpu.VMEM((CHUNK, 16), F32),
        pltpu.VMEM((CHUNK,), F32),
        pltpu.VMEM((RPT // 32, 16), F32),
        pltpu.VMEM_SHARED((NPAD, 16), F32),
        pltpu.SemaphoreType.DMA,
        pltpu.SemaphoreType.DMA,
    ]
    if with_ex:
        @functools.partial(
            pl.kernel,
            out_type=jax.ShapeDtypeStruct((ncols * NPAD, 16), F32),
            mesh=plsc.VectorSubcoreMesh(**_MESH),
            compiler_params=_SC_PARAMS,
            scratch_types=scratch,
        )
        def k(ytab, src1, dst1, ex1, out, *s):
            _rows_body(True, rounds, ytab, src1, dst1, ex1, out, *s)
    else:
        @functools.partial(
            pl.kernel,
            out_type=jax.ShapeDtypeStruct((ncols * NPAD, 16), F32),
            mesh=plsc.VectorSubcoreMesh(**_MESH),
            compiler_params=_SC_PARAMS,
            scratch_types=scratch,
        )
        def k(ytab, src1, dst1, out, *s):
            _rows_body(False, rounds, ytab, src1, dst1, None, out, *s)
    return k


_k_rows32 = _make_rows(False, 1)
_k_rows = _make_rows(False, 2)
_k_rows_ex = _make_rows(True, 2)


def _gatsc_body(als, ald, cs, src1, dst1, ex1, dpart, srcv, dstv, asv, adv,
                csv, exv, zv, shared, gsem, ssem):
    core = lax.axis_index("c")
    sub = lax.axis_index("s")
    wid = core * 16 + sub
    row0 = sub * RPT
    _zero_vmem_1d(zv, RPT // 16)
    pltpu.sync_copy(zv, shared.at[pl.ds(row0, RPT)])
    plsc.subcore_barrier()

    def chunk(j, _):
        e0 = wid * EPW + j * CHUNK
        pltpu.sync_copy(src1.at[pl.ds(e0, CHUNK)], srcv)
        pltpu.sync_copy(dst1.at[pl.ds(e0, CHUNK)], dstv)
        gd = [
            pltpu.async_copy(als.at[srcv], asv, gsem),
            pltpu.async_copy(ald.at[dstv], adv, gsem),
            pltpu.async_copy(cs.at[dstv], csv, gsem),
        ]
        for d in gd:
            d.wait()

        def comp(i, _):
            sl = pl.ds(i * 16, 16)
            s = asv[sl] + adv[sl]
            e = jnp.maximum(s, 0.0) + 0.2 * jnp.minimum(s, 0.0)
            exv[sl] = jnp.exp(e - csv[sl])
            return 0

        lax.fori_loop(0, CHUNK // 16, comp, 0)
        pltpu.sync_copy(exv, ex1.at[pl.ds(e0, CHUNK)])
        pltpu.async_copy(exv, shared.at[dstv], ssem, add=True).wait()
        return 0

    lax.fori_loop(0, EPW // CHUNK, chunk, 0)
    plsc.subcore_barrier()
    pltpu.async_copy(shared.at[pl.ds(row0, RPT)],
                     dpart.at[core, pl.ds(row0, RPT)], gsem).wait()


@functools.partial(
    pl.kernel,
    out_type=(jax.ShapeDtypeStruct((EPAD,), F32),
              jax.ShapeDtypeStruct((2, NPAD), F32)),
    mesh=plsc.VectorSubcoreMesh(**_MESH),
    compiler_params=_SC_PARAMS,
    scratch_types=[
        pltpu.VMEM((CHUNK,), I32),
        pltpu.VMEM((CHUNK,), I32),
        pltpu.VMEM((CHUNK,), F32),
        pltpu.VMEM((CHUNK,), F32),
        pltpu.VMEM((CHUNK,), F32),
        pltpu.VMEM((CHUNK,), F32),
        pltpu.VMEM((RPT,), F32),
        pltpu.VMEM_SHARED((NPAD,), F32),
        pltpu.SemaphoreType.DMA,
        pltpu.SemaphoreType.DMA,
    ],
)
def _k_gatsc(als, ald, cs, src1, dst1, ex1, dpart, *s):
    _gatsc_body(als, ald, cs, src1, dst1, ex1, dpart, *s)


def _prep1_body(x_ref, degp_ref, ytab_ref, dinv_ref):
    deg = degp_ref[0] + degp_ref[1] + 1.0
    dinv = lax.rsqrt(deg)
    y = dinv * x_ref[...]
    for c in range(2):
        ytab_ref[c] = y[:, c * 16:(c + 1) * 16]
    dinv_ref[...] = dinv


_k_prep1 = pl.pallas_call(
    _prep1_body,
    grid=(NB,),
    in_specs=[
        pl.BlockSpec((BLK, 32), lambda i: (i, 0)),
        pl.BlockSpec((2, BLK, 1), lambda i: (0, i, 0)),
    ],
    out_specs=[
        pl.BlockSpec((2, BLK, 16), lambda i: (0, i, 0)),
        pl.BlockSpec((BLK, 1), lambda i: (i, 0)),
    ],
    out_shape=[
        jax.ShapeDtypeStruct((2, NPAD, 16), F32),
        jax.ShapeDtypeStruct((NPAD, 1), F32),
    ],
)


def _gcn1_body(raw_ref, x_ref, dinv_ref, w1_ref, b1_ref, w2_ref, ytab_ref):
    dinv = dinv_ref[...]
    inner = jnp.concatenate([raw_ref[0], raw_ref[1]], axis=1)
    inner = dinv * (inner + dinv * x_ref[...])
    h1 = jnp.maximum(
        jnp.dot(inner, w1_ref[...], preferred_element_type=F32)
        + b1_ref[...], 0.0)
    y = dinv * jnp.dot(h1, w2_ref[...], preferred_element_type=F32)
    for c in range(4):
        ytab_ref[c] = y[:, c * 16:(c + 1) * 16]


_k_gcn1 = pl.pallas_call(
    _gcn1_body,
    grid=(NB,),
    in_specs=[
        pl.BlockSpec((2, BLK, 16), lambda i: (0, i, 0)),
        pl.BlockSpec((BLK, 32), lambda i: (i, 0)),
        pl.BlockSpec((BLK, 1), lambda i: (i, 0)),
        pl.BlockSpec((32, 64), lambda i: (0, 0)),
        pl.BlockSpec((64,), lambda i: (0,)),
        pl.BlockSpec((64, 64), lambda i: (0, 0)),
    ],
    out_specs=pl.BlockSpec((4, BLK, 16), lambda i: (0, i, 0)),
    out_shape=jax.ShapeDtypeStruct((4, NPAD, 16), F32),
)


def _gatprep_body(scale_kind, raw_ref, selftab_ref, s_ref, b_ref, w_ref,
                  asrc_ref, adst_ref, xwtab_ref, als_ref, ald_ref, cs_ref):
    if scale_kind == "dinv":
        scale = s_ref[...]
    else:
        scale = 1.0 / (s_ref[0] + s_ref[1] + 1.0)
    acc = jnp.zeros((BLK, 64), F32)
    for c in range(4):
        pre = raw_ref[c] + selftab_ref[c]
        hc = jnp.maximum(scale * pre + b_ref[pl.ds(c * 16, 16)], 0.0)
        acc = acc + jnp.dot(hc, w_ref[pl.ds(c * 16, 16), :],
                            preferred_element_type=F32)
    for c in range(4):
        xwtab_ref[c] = acc[:, c * 16:(c + 1) * 16]
    als = jnp.dot(acc, asrc_ref[...], preferred_element_type=F32)
    ald = jnp.dot(acc, adst_ref[...], preferred_element_type=F32)
    s = als + ald
    cs = jnp.maximum(s, 0.0) + 0.2 * jnp.minimum(s, 0.0)
    als_ref[...] = als
    ald_ref[...] = ald
    cs_ref[...] = cs


def _make_gatprep(scale_kind):
    sspec = (pl.BlockSpec((BLK, 1), lambda i: (i, 0)) if scale_kind == "dinv"
             else pl.BlockSpec((2, BLK, 1), lambda i: (0, i, 0)))
    return pl.pallas_call(
        functools.partial(_gatprep_body, scale_kind),
        grid=(NB,),
        in_specs=[
            pl.BlockSpec((4, BLK, 16), lambda i: (0, i, 0)),
            pl.BlockSpec((4, BLK, 16), lambda i: (0, i, 0)),
            sspec,
            pl.BlockSpec((64,), lambda i: (0,)),
            pl.BlockSpec((64, 64), lambda i: (0, 0)),
            pl.BlockSpec((64, 1), lambda i: (0, 0)),
            pl.BlockSpec((64, 1), lambda i: (0, 0)),
        ],
        out_specs=[
            pl.BlockSpec((4, BLK, 16), lambda i: (0, i, 0)),
            pl.BlockSpec((BLK, 1), lambda i: (i, 0)),
            pl.BlockSpec((BLK, 1), lambda i: (i, 0)),
            pl.BlockSpec((BLK, 1), lambda i: (i, 0)),
        ],
        out_shape=[
            jax.ShapeDtypeStruct((4, NPAD, 16), F32),
            jax.ShapeDtypeStruct((NPAD, 1), F32),
            jax.ShapeDtypeStruct((NPAD, 1), F32),
            jax.ShapeDtypeStruct((NPAD, 1), F32),
        ],
    )


_k_gatprep_dinv = _make_gatprep("dinv")
_k_gatprep_denom = _make_gatprep("denom")


def _final_body(raw_ref, selftab_ref, dp_ref, b_ref, batch_ref, wf1_ref,
                bf1_ref, wf2_ref, bf2_ref, out_ref, sums_ref, cnt_ref):
    i = pl.program_id(0)

    @pl.when(i == 0)
    def _():
        sums_ref[...] = jnp.zeros((G, 64), F32)
        cnt_ref[...] = jnp.zeros((G, 1), F32)

    scale = 1.0 / (dp_ref[0] + dp_ref[1] + 1.0)
    hs = []
    for c in range(4):
        pre = raw_ref[c] + selftab_ref[c]
        hs.append(jnp.maximum(scale * pre + b_ref[pl.ds(c * 16, 16)], 0.0))
    h = jnp.concatenate(hs, axis=1)
    gids = lax.broadcasted_iota(I32, (1, G), 1)
    oh = (batch_ref[...] == gids).astype(F32)
    sums_ref[...] += lax.dot_general(oh, h, (((0,), (0,)), ((), ())),
                                     preferred_element_type=F32)
    cnt_ref[...] += lax.dot_general(oh, jnp.ones((BLK, 1), F32),
                                    (((0,), (0,)), ((), ())),
                                    preferred_element_type=F32)

    @pl.when(i == NB - 1)
    def _():
        pooled = sums_ref[...] / jnp.maximum(cnt_ref[...], 1.0)
        hf = jnp.maximum(
            jnp.dot(pooled, wf1_ref[...], preferred_element_type=F32)
            + bf1_ref[...], 0.0)
        out_ref[...] = (jnp.dot(hf, wf2_ref[...], preferred_element_type=F32)
                        + bf2_ref[...])


_k_final = pl.pallas_call(
    _final_body,
    grid=(NB,),
    in_specs=[
        pl.BlockSpec((4, BLK, 16), lambda i: (0, i, 0)),
        pl.BlockSpec((4, BLK, 16), lambda i: (0, i, 0)),
        pl.BlockSpec((2, BLK, 1), lambda i: (0, i, 0)),
        pl.BlockSpec((64,), lambda i: (0,)),
        pl.BlockSpec((BLK, 1), lambda i: (i, 0)),
        pl.BlockSpec((64, 32), lambda i: (0, 0)),
        pl.BlockSpec((32,), lambda i: (0,)),
        pl.BlockSpec((32, 1), lambda i: (0, 0)),
        pl.BlockSpec((1,), lambda i: (0,)),
    ],
    out_specs=pl.BlockSpec((G, 1), lambda i: (0, 0)),
    out_shape=jax.ShapeDtypeStruct((G, 1), F32),
    scratch_shapes=[
        pltpu.VMEM((G, 64), F32),
        pltpu.VMEM((G, 1), F32),
    ],
)


def kernel(x, edge_index, edge_attr, batch, W1, b1, W2, b2, Wg1, asrc1,
           adst1, bg1, Wg2, asrc2, adst2, bg2, Wf1, bf1, Wf2, bf2):
    del edge_attr
    npad_e = EPAD - edge_index.shape[1]
    padi = jnp.full((npad_e,), N, I32)
    src1 = jnp.concatenate([edge_index[0], padi])
    dst1 = jnp.concatenate([edge_index[1], padi])
    x_pad = jnp.zeros((NPAD, 32), F32).at[:N].set(x)
    batch_pad = jnp.full((NPAD, 1), -1, I32).at[:N, 0].set(batch)

    degp = _k_deg(dst1)
    ytab1, dinv = _k_prep1(x_pad, degp.reshape(2, NPAD, 1))
    raw1 = _k_rows32(ytab1.reshape(2 * NPAD, 16), src1, dst1)
    ytab2 = _k_gcn1(raw1.reshape(2, NPAD, 16), x_pad, dinv, W1, b1, W2)
    raw2 = _k_rows(ytab2.reshape(4 * NPAD, 16), src1, dst1)
    xwtab3, als3, ald3, cs3 = _k_gatprep_dinv(
        raw2.reshape(4, NPAD, 16), ytab2, dinv, b2, Wg1, asrc1.reshape(64, 1),
        adst1.reshape(64, 1))
    ex3, dp3 = _k_gatsc(als3.reshape(NPAD), ald3.reshape(NPAD),
                        cs3.reshape(NPAD), src1, dst1)
    raw3 = _k_rows_ex(xwtab3.reshape(4 * NPAD, 16), src1, dst1, ex3)
    xwtab4, als4, ald4, cs4 = _k_gatprep_denom(
        raw3.reshape(4, NPAD, 16), xwtab3, dp3.reshape(2, NPAD, 1), bg1, Wg2,
        asrc2.reshape(64, 1), adst2.reshape(64, 1))
    ex4, dp4 = _k_gatsc(als4.reshape(NPAD), ald4.reshape(NPAD),
                        cs4.reshape(NPAD), src1, dst1)
    raw4 = _k_rows_ex(xwtab4.reshape(4 * NPAD, 16), src1, dst1, ex4)
    out = _k_final(raw4.reshape(4, NPAD, 16), xwtab4, dp4.reshape(2, NPAD, 1),
                   bg2, batch_pad, Wf1, bf1, Wf2, bf2)
    return out.reshape(G)
